# baseline (speedup 1.0000x reference)
import functools

import jax
import jax.numpy as jnp
from jax import lax
from jax.experimental import pallas as pl
from jax.experimental.pallas import tpu as pltpu
from jax.experimental.pallas import tpu_sc as plsc

NN = 10000
DD = 8
EE = 320000
D2 = DD * DD
NW = 32
EPW = EE // NW
CH = 80
NCH = EPW // CH
BE = 1000
NEB = EE // BE
NNB = NN // BE


def _mesh():
    return plsc.VectorSubcoreMesh(core_axis_name="c", subcore_axis_name="s")


def _sc_gather_maps(table, sihash, pos, rhash):
    @functools.partial(
        pl.kernel,
        mesh=_mesh(),
        compiler_params=pltpu.CompilerParams(use_tc_tiling_on_sc=False,
                                             needs_layout_passes=False),
        out_type=jax.ShapeDtypeStruct((EE, D2), jnp.float32),
        scratch_types=[
            pltpu.VMEM((CH,), jnp.int32),
            pltpu.VMEM((CH,), jnp.int32),
            pltpu.VMEM((CH, DD), jnp.int32),
            pltpu.VMEM((CH,), jnp.int32),
            pltpu.VMEM((CH, D2), jnp.float32),
            pltpu.SemaphoreType.DMA,
            pltpu.SemaphoreType.DMA,
        ],
    )
    def k(table_hbm, sihash_hbm, pos_hbm, rhash_hbm, out_hbm,
          pos_v, rh_v, sih_v, sel_v, rows_v, sem_a, sem_b):
        wid = lax.axis_index("s") * 2 + lax.axis_index("c")
        base = wid * EPW

        def body(i, carry):
            off = pl.multiple_of(base + i * CH, 8)
            pltpu.sync_copy(pos_hbm.at[pl.ds(off, CH)], pos_v)
            pltpu.sync_copy(rhash_hbm.at[pl.ds(off, CH)], rh_v)
            pltpu.async_copy(sihash_hbm.at[pos_v], sih_v, sem_a).wait()
            lanes = lax.iota(jnp.int32, 16)
            for j in range(CH // 16):
                rows = j * 16 + lanes
                si = plsc.load_gather(sih_v, [rows, lanes * 0])
                sh = plsc.load_gather(sih_v, [rows, lanes * 0 + 1])
                rh = rh_v[pl.ds(j * 16, 16)]
                eid = off + j * 16 + lanes
                sel_v[pl.ds(j * 16, 16)] = jnp.where(sh == rh, si, eid)
            pltpu.async_copy(table_hbm.at[sel_v], rows_v, sem_b).wait()
            pltpu.sync_copy(rows_v, out_hbm.at[pl.ds(off, CH)])
            return carry

        lax.fori_loop(0, NCH, body, 0)

    return k(table, sihash, pos, rhash)


def _sc_degree(rs, row, zeros_nn8):
    @functools.partial(
        pl.kernel,
        mesh=_mesh(),
        compiler_params=pltpu.CompilerParams(use_tc_tiling_on_sc=False),
        out_type=jax.ShapeDtypeStruct((2, NN, DD), jnp.float32),
        scratch_types=[
            pltpu.VMEM((CH,), jnp.int32),
            pltpu.VMEM((CH, DD), jnp.float32),
            pltpu.VMEM_SHARED((NN, DD), jnp.float32),
            pltpu.SemaphoreType.DMA,
        ],
    )
    def k(rs_hbm, row_hbm, zero_hbm, out_hbm, idx_v, val_v, deg_sh, sem):
        c = lax.axis_index("c")
        s = lax.axis_index("s")
        wid = s * 2 + c
        base = wid * EPW

        @pl.when(s < 10)
        def _zero():
            off = pl.multiple_of(s * BE, 8)
            pltpu.sync_copy(zero_hbm.at[pl.ds(off, BE)], deg_sh.at[pl.ds(off, BE)])

        plsc.subcore_barrier()

        def body(i, carry):
            off = pl.multiple_of(base + i * CH, 8)
            pltpu.sync_copy(row_hbm.at[pl.ds(off, CH)], idx_v)
            pltpu.sync_copy(rs_hbm.at[pl.ds(off, CH)], val_v)
            pltpu.async_copy(val_v, deg_sh.at[idx_v], sem, add=True).wait()
            return carry

        lax.fori_loop(0, NCH, body, 0)
        plsc.subcore_barrier()

        @pl.when(s < 10)
        def _flush():
            off = pl.multiple_of(s * BE, 8)
            pltpu.sync_copy(deg_sh.at[pl.ds(off, BE)], out_hbm.at[c, pl.ds(off, BE)])

    return k(rs, row, zeros_nn8)


def _sc_gather_dinv(dinv, row, col):
    @functools.partial(
        pl.kernel,
        mesh=_mesh(),
        compiler_params=pltpu.CompilerParams(use_tc_tiling_on_sc=False),
        out_type=(
            jax.ShapeDtypeStruct((EE, DD), jnp.float32),
            jax.ShapeDtypeStruct((EE, DD), jnp.float32),
        ),
        scratch_types=[
            pltpu.VMEM((CH,), jnp.int32),
            pltpu.VMEM((CH,), jnp.int32),
            pltpu.VMEM((CH, DD), jnp.float32),
            pltpu.VMEM((CH, DD), jnp.float32),
            pltpu.SemaphoreType.DMA,
            pltpu.SemaphoreType.DMA,
        ],
    )
    def k(dinv_hbm, row_hbm, col_hbm, dr_hbm, dc_hbm,
          ridx_v, cidx_v, dr_v, dc_v, sem_r, sem_c):
        wid = lax.axis_index("s") * 2 + lax.axis_index("c")
        base = wid * EPW

        def body(i, carry):
            off = pl.multiple_of(base + i * CH, 8)
            pltpu.sync_copy(row_hbm.at[pl.ds(off, CH)], ridx_v)
            pltpu.sync_copy(col_hbm.at[pl.ds(off, CH)], cidx_v)
            cp_r = pltpu.async_copy(dinv_hbm.at[ridx_v], dr_v, sem_r)
            cp_c = pltpu.async_copy(dinv_hbm.at[cidx_v], dc_v, sem_c)
            cp_r.wait()
            cp_c.wait()
            pltpu.sync_copy(dr_v, dr_hbm.at[pl.ds(off, CH)])
            pltpu.sync_copy(dc_v, dc_hbm.at[pl.ds(off, CH)])
            return carry

        lax.fori_loop(0, NCH, body, 0)

    return k(dinv, row, col)


def _expanders():
    lane = lax.broadcasted_iota(jnp.int32, (DD, D2), 1)
    ridx = lax.broadcasted_iota(jnp.int32, (DD, D2), 0)
    ea = (lane // DD == ridx).astype(jnp.float32)
    ec = (lane % DD == ridx).astype(jnp.float32)
    return ea, ec


def _tc_bmm_body(s_ref, t_ref, p_ref, rs_ref):
    ea, ec = _expanders()
    s = s_ref[...]
    t = t_ref[...]
    acc = jnp.zeros((BE, D2), jnp.float32)
    for b in range(DD):
        sb = s[:, b * DD:(b + 1) * DD]
        tb = t[:, b * DD:(b + 1) * DD]
        acc = acc + (lax.dot(sb, ea, preferred_element_type=jnp.float32)
                     * lax.dot(tb, ec, preferred_element_type=jnp.float32))
    p_ref[...] = acc
    lane = lax.broadcasted_iota(jnp.int32, (D2, DD), 0)
    aidx = lax.broadcasted_iota(jnp.int32, (D2, DD), 1)
    eat = (lane // DD == aidx).astype(jnp.float32)
    rs_ref[...] = lax.dot(jnp.abs(acc), eat, preferred_element_type=jnp.float32)


def _tc_bmm(f2, fd2):
    return pl.pallas_call(
        _tc_bmm_body,
        grid=(NEB,),
        in_specs=[
            pl.BlockSpec((BE, D2), lambda i: (i, 0)),
            pl.BlockSpec((BE, D2), lambda i: (i, 0)),
        ],
        out_specs=[
            pl.BlockSpec((BE, D2), lambda i: (i, 0)),
            pl.BlockSpec((BE, DD), lambda i: (i, 0)),
        ],
        out_shape=[
            jax.ShapeDtypeStruct((EE, D2), jnp.float32),
            jax.ShapeDtypeStruct((EE, DD), jnp.float32),
        ],
    )(f2, fd2)


def _tc_finalize_body(p_ref, dr_ref, dc_ref, row_ref, col_ref, dinv_ref,
                      vals_ref, ind_ref):
    i = pl.program_id(0)
    ea, ec = _expanders()
    lane = lax.broadcasted_iota(jnp.int32, (BE, D2), 1)
    a_ln = lane // DD
    c_ln = lane % DD

    @pl.when(i < NEB)
    def _edges():
        dre = lax.dot(dr_ref[...], ea, preferred_element_type=jnp.float32)
        dce = lax.dot(dc_ref[...], ec, preferred_element_type=jnp.float32)
        vals_ref[0] = dre * p_ref[...] * dce
        r = row_ref[...]
        cc = col_ref[...]
        ind_ref[0, 0] = r * DD + a_ln
        ind_ref[1, 0] = cc * DD + c_ln

    @pl.when(i >= NEB)
    def _diag():
        dv = dinv_ref[...]
        de_a = lax.dot(dv, ea, preferred_element_type=jnp.float32)
        de_c = lax.dot(dv, ec, preferred_element_type=jnp.float32)
        vals_ref[0] = jnp.where(a_ln == c_ln, de_a * de_c, 0.0)
        nid = (i - NEB) * BE + lax.broadcasted_iota(jnp.int32, (BE, D2), 0)
        ind_ref[0, 0] = nid * DD + a_ln
        ind_ref[1, 0] = nid * DD + c_ln


def _tc_finalize(p, dr, dc, row2, col2, dinv):
    nblk = NEB + NNB
    vals, inds = pl.pallas_call(
        _tc_finalize_body,
        grid=(nblk,),
        in_specs=[
            pl.BlockSpec((BE, D2), lambda i: (jnp.minimum(i, NEB - 1), 0)),
            pl.BlockSpec((BE, DD), lambda i: (jnp.minimum(i, NEB - 1), 0)),
            pl.BlockSpec((BE, DD), lambda i: (jnp.minimum(i, NEB - 1), 0)),
            pl.BlockSpec((BE, 1), lambda i: (jnp.minimum(i, NEB - 1), 0)),
            pl.BlockSpec((BE, 1), lambda i: (jnp.minimum(i, NEB - 1), 0)),
            pl.BlockSpec((BE, DD), lambda i: (jnp.maximum(i - NEB, 0), 0)),
        ],
        out_specs=[
            pl.BlockSpec((1, BE, D2), lambda i: (i, 0, 0)),
            pl.BlockSpec((2, 1, BE, D2), lambda i: (0, i, 0, 0)),
        ],
        out_shape=[
            jax.ShapeDtypeStruct((nblk, BE, D2), jnp.float32),
            jax.ShapeDtypeStruct((2, nblk, BE, D2), jnp.int32),
        ],
    )(p, dr, dc, row2, col2, dinv)
    return vals, inds


def kernel(restriction_maps, edge_index):
    row = edge_index[0]
    col = edge_index[1]
    edge_hash = row * NN + col
    reverse_hash = col * NN + row
    sorted_hash, sort_idx = lax.sort(
        (edge_hash, jnp.arange(EE, dtype=jnp.int32)), num_keys=1, is_stable=True)
    pos = jnp.searchsorted(sorted_hash, reverse_hash)
    pos = jnp.clip(pos, 0, EE - 1)
    sihash = jnp.concatenate(
        [sort_idx[:, None], sorted_hash[:, None],
         jnp.zeros((EE, DD - 2), jnp.int32)], axis=1)

    f2 = restriction_maps.reshape(EE, D2)
    fd2 = _sc_gather_maps(f2, sihash, pos, reverse_hash)
    p, rs = _tc_bmm(f2, fd2)
    zeros_nn8 = jnp.zeros((NN, DD), jnp.float32)
    deg_part = _sc_degree(rs, row, zeros_nn8)
    deg = deg_part[0] + deg_part[1] + 1.0
    dinv = jnp.where(deg > 0, deg ** -0.5, 0.0)
    dr, dc = _sc_gather_dinv(dinv, row, col)
    vals4, inds = _tc_finalize(p, dr, dc,
                               row.reshape(EE, 1), col.reshape(EE, 1), dinv)
    indices = inds.reshape(2, (EE + NN) * D2)
    vals = vals4.reshape((EE + NN) * D2)
    transport_maps = p.reshape(EE, DD, DD)
    return indices, vals, transport_maps


# speedup vs baseline: 1.4523x; 1.4523x over previous
import functools

import jax
import jax.numpy as jnp
from jax import lax
from jax.experimental import pallas as pl
from jax.experimental.pallas import tpu as pltpu
from jax.experimental.pallas import tpu_sc as plsc

NN = 10000
DD = 8
EE = 320000
D2 = DD * DD
NW = 32
EPW = EE // NW
CH = 80
NCH = EPW // CH
BE = 1000
NEB = EE // BE
NNB = NN // BE


def _mesh():
    return plsc.VectorSubcoreMesh(core_axis_name="c", subcore_axis_name="s")


RUN = 32
NRUN = EE // RUN


def _sc_gather_maps(table, sihash, shash2, tops, rhash):
    @functools.partial(
        pl.kernel,
        mesh=_mesh(),
        compiler_params=pltpu.CompilerParams(use_tc_tiling_on_sc=False,
                                             needs_layout_passes=False),
        out_type=jax.ShapeDtypeStruct((EE, D2), jnp.float32),
        scratch_types=[
            pltpu.VMEM((NRUN,), jnp.int32),
            pltpu.VMEM((CH,), jnp.int32),
            pltpu.VMEM((CH,), jnp.int32),
            pltpu.VMEM((CH, RUN), jnp.int32),
            pltpu.VMEM((CH,), jnp.int32),
            pltpu.VMEM((CH, DD), jnp.int32),
            pltpu.VMEM((CH,), jnp.int32),
            pltpu.VMEM((CH, D2), jnp.float32),
            pltpu.SemaphoreType.DMA,
        ],
    )
    def k(table_hbm, sihash_hbm, shash2_hbm, tops_hbm, rhash_hbm, out_hbm,
          tops_v, rh_v, run_v, runs_v, pos_v, sih_v, sel_v, rows_v, sem):
        wid = lax.axis_index("s") * 2 + lax.axis_index("c")
        base = wid * EPW
        pltpu.sync_copy(tops_hbm, tops_v)
        lanes = lax.iota(jnp.int32, 16)

        def lower_bound(ref, extra_idx, q, lo0, hi0, rounds):
            lo = lanes * 0 + lo0
            hi = lanes * 0 + hi0
            for _ in range(rounds):
                mid = (lo + hi) >> 1
                v = plsc.load_gather(ref, extra_idx + [mid])
                pred = v < q
                lo = jnp.where(pred, mid + 1, lo)
                hi = jnp.where(pred, hi, mid)
            return lo

        def body(i, carry):
            off = pl.multiple_of(base + i * CH, 8)
            pltpu.sync_copy(rhash_hbm.at[pl.ds(off, CH)], rh_v)
            for j in range(CH // 16):
                q = rh_v[pl.ds(j * 16, 16)]
                t = lower_bound(tops_v, [], q, 0, NRUN, 14)
                run_v[pl.ds(j * 16, 16)] = jnp.maximum(t - 1, 0)
            pltpu.async_copy(shash2_hbm.at[run_v], runs_v, sem).wait()
            for j in range(CH // 16):
                q = rh_v[pl.ds(j * 16, 16)]
                rows = j * 16 + lanes
                r = run_v[pl.ds(j * 16, 16)]
                tv = plsc.load_gather(tops_v, [r])
                c1 = lower_bound(runs_v, [rows], q, 1, RUN, 5)
                c = jnp.where(tv < q, c1, 0)
                pos_v[pl.ds(j * 16, 16)] = jnp.minimum(r * RUN + c, EE - 1)
            pltpu.async_copy(sihash_hbm.at[pos_v], sih_v, sem).wait()
            for j in range(CH // 16):
                rows = j * 16 + lanes
                si = plsc.load_gather(sih_v, [rows, lanes * 0])
                sh = plsc.load_gather(sih_v, [rows, lanes * 0 + 1])
                rh = rh_v[pl.ds(j * 16, 16)]
                eid = off + j * 16 + lanes
                sel_v[pl.ds(j * 16, 16)] = jnp.where(sh == rh, si, eid)
            pltpu.async_copy(table_hbm.at[sel_v], rows_v, sem).wait()
            pltpu.sync_copy(rows_v, out_hbm.at[pl.ds(off, CH)])
            return carry

        lax.fori_loop(0, NCH, body, 0)

    return k(table, sihash, shash2, tops, rhash)


def _sc_degree(rs, row, zeros_nn8):
    @functools.partial(
        pl.kernel,
        mesh=_mesh(),
        compiler_params=pltpu.CompilerParams(use_tc_tiling_on_sc=False),
        out_type=jax.ShapeDtypeStruct((2, NN, DD), jnp.float32),
        scratch_types=[
            pltpu.VMEM((CH,), jnp.int32),
            pltpu.VMEM((CH, DD), jnp.float32),
            pltpu.VMEM_SHARED((NN, DD), jnp.float32),
            pltpu.SemaphoreType.DMA,
        ],
    )
    def k(rs_hbm, row_hbm, zero_hbm, out_hbm, idx_v, val_v, deg_sh, sem):
        c = lax.axis_index("c")
        s = lax.axis_index("s")
        wid = s * 2 + c
        base = wid * EPW

        @pl.when(s < 10)
        def _zero():
            off = pl.multiple_of(s * BE, 8)
            pltpu.sync_copy(zero_hbm.at[pl.ds(off, BE)], deg_sh.at[pl.ds(off, BE)])

        plsc.subcore_barrier()

        def body(i, carry):
            off = pl.multiple_of(base + i * CH, 8)
            pltpu.sync_copy(row_hbm.at[pl.ds(off, CH)], idx_v)
            pltpu.sync_copy(rs_hbm.at[pl.ds(off, CH)], val_v)
            pltpu.async_copy(val_v, deg_sh.at[idx_v], sem, add=True).wait()
            return carry

        lax.fori_loop(0, NCH, body, 0)
        plsc.subcore_barrier()

        @pl.when(s < 10)
        def _flush():
            off = pl.multiple_of(s * BE, 8)
            pltpu.sync_copy(deg_sh.at[pl.ds(off, BE)], out_hbm.at[c, pl.ds(off, BE)])

    return k(rs, row, zeros_nn8)


def _sc_gather_dinv(dinv, row, col):
    @functools.partial(
        pl.kernel,
        mesh=_mesh(),
        compiler_params=pltpu.CompilerParams(use_tc_tiling_on_sc=False),
        out_type=(
            jax.ShapeDtypeStruct((EE, DD), jnp.float32),
            jax.ShapeDtypeStruct((EE, DD), jnp.float32),
        ),
        scratch_types=[
            pltpu.VMEM((CH,), jnp.int32),
            pltpu.VMEM((CH,), jnp.int32),
            pltpu.VMEM((CH, DD), jnp.float32),
            pltpu.VMEM((CH, DD), jnp.float32),
            pltpu.SemaphoreType.DMA,
            pltpu.SemaphoreType.DMA,
        ],
    )
    def k(dinv_hbm, row_hbm, col_hbm, dr_hbm, dc_hbm,
          ridx_v, cidx_v, dr_v, dc_v, sem_r, sem_c):
        wid = lax.axis_index("s") * 2 + lax.axis_index("c")
        base = wid * EPW

        def body(i, carry):
            off = pl.multiple_of(base + i * CH, 8)
            pltpu.sync_copy(row_hbm.at[pl.ds(off, CH)], ridx_v)
            pltpu.sync_copy(col_hbm.at[pl.ds(off, CH)], cidx_v)
            cp_r = pltpu.async_copy(dinv_hbm.at[ridx_v], dr_v, sem_r)
            cp_c = pltpu.async_copy(dinv_hbm.at[cidx_v], dc_v, sem_c)
            cp_r.wait()
            cp_c.wait()
            pltpu.sync_copy(dr_v, dr_hbm.at[pl.ds(off, CH)])
            pltpu.sync_copy(dc_v, dc_hbm.at[pl.ds(off, CH)])
            return carry

        lax.fori_loop(0, NCH, body, 0)

    return k(dinv, row, col)


def _expanders():
    lane = lax.broadcasted_iota(jnp.int32, (DD, D2), 1)
    ridx = lax.broadcasted_iota(jnp.int32, (DD, D2), 0)
    ea = (lane // DD == ridx).astype(jnp.float32)
    ec = (lane % DD == ridx).astype(jnp.float32)
    return ea, ec


def _tc_bmm_body(s_ref, t_ref, p_ref, rs_ref):
    ea, ec = _expanders()
    s = s_ref[...]
    t = t_ref[...]
    acc = jnp.zeros((BE, D2), jnp.float32)
    for b in range(DD):
        sb = s[:, b * DD:(b + 1) * DD]
        tb = t[:, b * DD:(b + 1) * DD]
        acc = acc + (lax.dot(sb, ea, preferred_element_type=jnp.float32)
                     * lax.dot(tb, ec, preferred_element_type=jnp.float32))
    p_ref[...] = acc
    lane = lax.broadcasted_iota(jnp.int32, (D2, DD), 0)
    aidx = lax.broadcasted_iota(jnp.int32, (D2, DD), 1)
    eat = (lane // DD == aidx).astype(jnp.float32)
    rs_ref[...] = lax.dot(jnp.abs(acc), eat, preferred_element_type=jnp.float32)


def _tc_bmm(f2, fd2):
    return pl.pallas_call(
        _tc_bmm_body,
        grid=(NEB,),
        in_specs=[
            pl.BlockSpec((BE, D2), lambda i: (i, 0)),
            pl.BlockSpec((BE, D2), lambda i: (i, 0)),
        ],
        out_specs=[
            pl.BlockSpec((BE, D2), lambda i: (i, 0)),
            pl.BlockSpec((BE, DD), lambda i: (i, 0)),
        ],
        out_shape=[
            jax.ShapeDtypeStruct((EE, D2), jnp.float32),
            jax.ShapeDtypeStruct((EE, DD), jnp.float32),
        ],
    )(f2, fd2)


def _tc_finalize_body(p_ref, dr_ref, dc_ref, row_ref, col_ref, dinv_ref,
                      vals_ref, ind_ref):
    i = pl.program_id(0)
    ea, ec = _expanders()
    lane = lax.broadcasted_iota(jnp.int32, (BE, D2), 1)
    a_ln = lane // DD
    c_ln = lane % DD

    @pl.when(i < NEB)
    def _edges():
        dre = lax.dot(dr_ref[...], ea, preferred_element_type=jnp.float32)
        dce = lax.dot(dc_ref[...], ec, preferred_element_type=jnp.float32)
        vals_ref[0] = dre * p_ref[...] * dce
        r = row_ref[...]
        cc = col_ref[...]
        ind_ref[0, 0] = r * DD + a_ln
        ind_ref[1, 0] = cc * DD + c_ln

    @pl.when(i >= NEB)
    def _diag():
        dv = dinv_ref[...]
        de_a = lax.dot(dv, ea, preferred_element_type=jnp.float32)
        de_c = lax.dot(dv, ec, preferred_element_type=jnp.float32)
        vals_ref[0] = jnp.where(a_ln == c_ln, de_a * de_c, 0.0)
        nid = (i - NEB) * BE + lax.broadcasted_iota(jnp.int32, (BE, D2), 0)
        ind_ref[0, 0] = nid * DD + a_ln
        ind_ref[1, 0] = nid * DD + c_ln


def _tc_finalize(p, dr, dc, row2, col2, dinv):
    nblk = NEB + NNB
    vals, inds = pl.pallas_call(
        _tc_finalize_body,
        grid=(nblk,),
        in_specs=[
            pl.BlockSpec((BE, D2), lambda i: (jnp.minimum(i, NEB - 1), 0)),
            pl.BlockSpec((BE, DD), lambda i: (jnp.minimum(i, NEB - 1), 0)),
            pl.BlockSpec((BE, DD), lambda i: (jnp.minimum(i, NEB - 1), 0)),
            pl.BlockSpec((BE, 1), lambda i: (jnp.minimum(i, NEB - 1), 0)),
            pl.BlockSpec((BE, 1), lambda i: (jnp.minimum(i, NEB - 1), 0)),
            pl.BlockSpec((BE, DD), lambda i: (jnp.maximum(i - NEB, 0), 0)),
        ],
        out_specs=[
            pl.BlockSpec((1, BE, D2), lambda i: (i, 0, 0)),
            pl.BlockSpec((2, 1, BE, D2), lambda i: (0, i, 0, 0)),
        ],
        out_shape=[
            jax.ShapeDtypeStruct((nblk, BE, D2), jnp.float32),
            jax.ShapeDtypeStruct((2, nblk, BE, D2), jnp.int32),
        ],
    )(p, dr, dc, row2, col2, dinv)
    return vals, inds


def kernel(restriction_maps, edge_index):
    row = edge_index[0]
    col = edge_index[1]
    edge_hash = row * NN + col
    reverse_hash = col * NN + row
    sorted_hash, sort_idx = lax.sort(
        (edge_hash, jnp.arange(EE, dtype=jnp.int32)), num_keys=1, is_stable=True)
    sihash = jnp.concatenate(
        [sort_idx[:, None], sorted_hash[:, None],
         jnp.zeros((EE, DD - 2), jnp.int32)], axis=1)
    shash2 = sorted_hash.reshape(NRUN, RUN)
    tops = shash2[:, 0]

    f2 = restriction_maps.reshape(EE, D2)
    fd2 = _sc_gather_maps(f2, sihash, shash2, tops, reverse_hash)
    p, rs = _tc_bmm(f2, fd2)
    zeros_nn8 = jnp.zeros((NN, DD), jnp.float32)
    deg_part = _sc_degree(rs, row, zeros_nn8)
    deg = deg_part[0] + deg_part[1] + 1.0
    dinv = jnp.where(deg > 0, deg ** -0.5, 0.0)
    dr, dc = _sc_gather_dinv(dinv, row, col)
    vals4, inds = _tc_finalize(p, dr, dc,
                               row.reshape(EE, 1), col.reshape(EE, 1), dinv)
    indices = inds.reshape(2, (EE + NN) * D2)
    vals = vals4.reshape((EE + NN) * D2)
    transport_maps = p.reshape(EE, DD, DD)
    return indices, vals, transport_maps
